# baseline (speedup 1.0000x reference)
import functools
import jax
import jax.numpy as jnp
from jax import lax
from jax.experimental import pallas as pl
from jax.experimental.pallas import tpu as pltpu
from jax.experimental.pallas import tpu_sc as plsc


def kernel(x, pos_table):
    batch, seq_len, d_model = x.shape
    n_rows = batch * seq_len
    xf = x.reshape(n_rows * d_model)
    pf = pos_table.reshape(-1)
    NW = 32
    rows_per_w = n_rows // NW
    CH = 32
    n_chunks = rows_per_w // CH
    chunk_elems = CH * d_model

    mesh = plsc.VectorSubcoreMesh(core_axis_name="c", subcore_axis_name="s")

    @functools.partial(
        pl.kernel,
        mesh=mesh,
        out_type=jax.ShapeDtypeStruct((n_rows * d_model,), jnp.float32),
        scratch_types=[
            pltpu.VMEM((chunk_elems,), jnp.float32),
            pltpu.VMEM((chunk_elems,), jnp.float32),
        ],
    )
    def k(x_hbm, pos_hbm, out_hbm, xv, pv):
        wid = lax.axis_index("s") * 2 + lax.axis_index("c")
        base_row = wid * rows_per_w
        s_base_row = base_row % seq_len

        def chunk_body(ci, carry):
            r0 = (base_row + ci * CH) * d_model
            p0 = (s_base_row + ci * CH) * d_model
            pltpu.sync_copy(x_hbm.at[pl.ds(r0, chunk_elems)], xv)
            pltpu.sync_copy(pos_hbm.at[pl.ds(p0, chunk_elems)], pv)

            def add_body(j, c2):
                o = j * 16
                xv[pl.ds(o, 16)] = xv[pl.ds(o, 16)] + pv[pl.ds(o, 16)]
                return c2

            lax.fori_loop(0, chunk_elems // 16, add_body, 0, unroll=8)
            pltpu.sync_copy(xv, out_hbm.at[pl.ds(r0, chunk_elems)])
            return carry

        lax.fori_loop(0, n_chunks, chunk_body, 0)

    out = k(xf, pf)
    return out.reshape(batch, seq_len, d_model)


# speedup vs baseline: 1.0769x; 1.0769x over previous
import functools
import jax
import jax.numpy as jnp
from jax import lax
from jax.experimental import pallas as pl
from jax.experimental.pallas import tpu as pltpu
from jax.experimental.pallas import tpu_sc as plsc


def kernel(x, pos_table):
    batch, seq_len, d_model = x.shape
    n_rows = batch * seq_len
    xf = x.reshape(n_rows * d_model)
    pf = pos_table.reshape(-1)
    NW = 32
    SR = seq_len // NW
    CH = 16
    n_sc = SR // CH
    T = n_sc * batch
    ce = CH * d_model

    mesh = plsc.VectorSubcoreMesh(core_axis_name="c", subcore_axis_name="s")

    @functools.partial(
        pl.kernel,
        mesh=mesh,
        out_type=jax.ShapeDtypeStruct((n_rows * d_model,), jnp.float32),
        scratch_types=[
            pltpu.VMEM((2, ce), jnp.float32),
            pltpu.VMEM((2, ce), jnp.float32),
            pltpu.SemaphoreType.DMA,
            pltpu.SemaphoreType.DMA,
            pltpu.SemaphoreType.DMA,
            pltpu.SemaphoreType.DMA,
            pltpu.SemaphoreType.DMA,
            pltpu.SemaphoreType.DMA,
        ],
    )
    def k(x_hbm, pos_hbm, out_hbm, xv, pv, xs0, xs1, ps0, ps1, os0, os1):
        wid = lax.axis_index("s") * 2 + lax.axis_index("c")
        s0 = wid * SR
        x_sems = (xs0, xs1)
        p_sems = (ps0, ps1)
        o_sems = (os0, os1)

        def x_off(t):
            sc, b = t // batch, t % batch
            return (b * seq_len + s0 + sc * CH) * d_model

        def p_off(sc):
            return (s0 + sc * CH) * d_model

        def x_copy(t):
            return pltpu.make_async_copy(
                x_hbm.at[pl.ds(x_off(t), ce)], xv.at[t % 2], x_sems[t % 2])

        def p_copy(sc):
            return pltpu.make_async_copy(
                pos_hbm.at[pl.ds(p_off(sc), ce)], pv.at[sc % 2], p_sems[sc % 2])

        def o_copy(t):
            return pltpu.make_async_copy(
                xv.at[t % 2], out_hbm.at[pl.ds(x_off(t), ce)], o_sems[t % 2])

        p_copy(0).start()
        x_copy(0).start()
        for t in range(T):
            sc = t // batch
            nxt = t + 1
            if nxt < T:
                if nxt >= 2:
                    o_copy(nxt - 2).wait()
                x_copy(nxt).start()
                if nxt % batch == 0 and nxt // batch < n_sc:
                    p_copy(nxt // batch).start()
            x_copy(t).wait()
            if t % batch == 0:
                p_copy(sc).wait()

            xb = xv.at[t % 2]
            pb = pv.at[sc % 2]

            def add_body(j, c):
                o = j * 16
                xb[pl.ds(o, 16)] = xb[pl.ds(o, 16)] + pb[pl.ds(o, 16)]
                return c

            lax.fori_loop(0, ce // 16, add_body, 0, unroll=8)
            o_copy(t).start()
        o_copy(T - 2).wait()
        o_copy(T - 1).wait()

    out = k(xf, pf)
    return out.reshape(batch, seq_len, d_model)


# speedup vs baseline: 1.5674x; 1.4554x over previous
import functools
import jax
import jax.numpy as jnp
from jax import lax
from jax.experimental import pallas as pl
from jax.experimental.pallas import tpu as pltpu
from jax.experimental.pallas import tpu_sc as plsc


def kernel(x, pos_table):
    batch, seq_len, d_model = x.shape
    n_rows = batch * seq_len
    xf = x.reshape(n_rows * d_model)
    pf = pos_table.reshape(-1)
    NW = 32
    SR = seq_len // NW
    CH = 16
    n_sc = SR // CH
    T = n_sc * batch
    ce = CH * d_model

    mesh = plsc.VectorSubcoreMesh(core_axis_name="c", subcore_axis_name="s")

    @functools.partial(
        pl.kernel,
        mesh=mesh,
        out_type=jax.ShapeDtypeStruct((n_rows * d_model,), jnp.float32),
        scratch_types=[
            pltpu.VMEM((2, ce), jnp.float32),
            pltpu.VMEM((2, ce), jnp.float32),
            pltpu.SemaphoreType.DMA,
            pltpu.SemaphoreType.DMA,
            pltpu.SemaphoreType.DMA,
            pltpu.SemaphoreType.DMA,
            pltpu.SemaphoreType.DMA,
            pltpu.SemaphoreType.DMA,
        ],
    )
    def k(x_hbm, pos_hbm, out_hbm, xv, pv, xs0, xs1, ps0, ps1, os0, os1):
        wid = lax.axis_index("s") * 2 + lax.axis_index("c")
        s0 = wid * SR
        x_sems = (xs0, xs1)
        p_sems = (ps0, ps1)
        o_sems = (os0, os1)

        def x_off(t):
            sc, b = t // batch, t % batch
            return (b * seq_len + s0 + sc * CH) * d_model

        def p_off(sc):
            return (s0 + sc * CH) * d_model

        def x_copy(t):
            return pltpu.make_async_copy(
                x_hbm.at[pl.ds(x_off(t), ce)], xv.at[t % 2], x_sems[t % 2])

        def p_copy(sc):
            return pltpu.make_async_copy(
                pos_hbm.at[pl.ds(p_off(sc), ce)], pv.at[sc % 2], p_sems[sc % 2])

        def o_copy(t):
            return pltpu.make_async_copy(
                xv.at[t % 2], out_hbm.at[pl.ds(x_off(t), ce)], o_sems[t % 2])

        p_copy(0).start()
        x_copy(0).start()
        for t in range(T):
            sc = t // batch
            nxt = t + 1
            if nxt < T:
                if nxt >= 2:
                    o_copy(nxt - 2).wait()
                x_copy(nxt).start()
                if nxt % batch == 0 and nxt // batch < n_sc:
                    p_copy(nxt // batch).start()
            x_copy(t).wait()
            if t % batch == 0:
                p_copy(sc).wait()

            xb = xv.at[t % 2]
            pb = pv.at[sc % 2]

            @plsc.parallel_loop(0, ce, step=16, unroll=8)
            def add_body(o):
                xb[pl.ds(o, 16)] = xb[pl.ds(o, 16)] + pb[pl.ds(o, 16)]
            o_copy(t).start()
        o_copy(T - 2).wait()
        o_copy(T - 1).wait()

    out = k(xf, pf)
    return out.reshape(batch, seq_len, d_model)


# speedup vs baseline: 8.6434x; 5.5145x over previous
import jax
import jax.numpy as jnp
from jax.experimental import pallas as pl
from jax.experimental.pallas import tpu as pltpu


def _add_kernel(x_ref, pos_ref, o_ref):
    o_ref[...] = x_ref[...] + pos_ref[...]


def kernel(x, pos_table):
    batch, seq_len, d_model = x.shape
    bs = 2048
    grid = (seq_len // bs, batch)
    return pl.pallas_call(
        _add_kernel,
        grid=grid,
        in_specs=[
            pl.BlockSpec((1, bs, d_model), lambda s, b: (b, s, 0)),
            pl.BlockSpec((bs, d_model), lambda s, b: (s, 0)),
        ],
        out_specs=pl.BlockSpec((1, bs, d_model), lambda s, b: (b, s, 0)),
        out_shape=jax.ShapeDtypeStruct(x.shape, x.dtype),
        compiler_params=pltpu.CompilerParams(vmem_limit_bytes=128 * 1024 * 1024),
    )(x, pos_table)
